# baseline (speedup 1.0000x reference)
import numpy as np

import jax
import jax.numpy as jnp
from jax.experimental import pallas as pl
from jax.experimental.pallas import tpu as pltpu

_BETA = 1000.0
_LOG2E = 1.4426950408889634
_N, _C, _H, _W = 4, 21, 512, 512
_BR = 256
_NRB = _H // _BR
_PH = 520

_CHS = 16
_NCS = _BR // _CHS
_CHC = 8
_NCC = _H // _CHC

_EPS = 1e-06


def _shift_cols(v, d):
    if d == 0:
        return v
    rows = v.shape[0]
    z = jnp.zeros((rows, abs(d)), jnp.float32)
    if d > 0:
        return jnp.concatenate([v[:, d:], z], axis=1)
    return jnp.concatenate([z, v[:, :d]], axis=1)


def _body(w_sref, seg_ref, out_ref, s_buf):
    n_idx = pl.program_id(0)
    rb = pl.program_id(1)
    scale = jnp.float32(_BETA * _LOG2E)

    @pl.when(rb == 0)
    def _zero_halo():
        s_buf[_H:_PH, :] = jnp.zeros((_PH - _H, _W), jnp.float32)

    for ch in range(_NCS):
        r0 = ch * _CHS
        mx = [seg_ref[0, c, r0:r0 + _CHS, :] for c in range(4)]
        for c in range(4, _C):
            mx[c % 4] = jnp.maximum(mx[c % 4], seg_ref[0, c, r0:r0 + _CHS, :])
        m = jnp.maximum(jnp.maximum(mx[0], mx[1]), jnp.maximum(mx[2], mx[3]))
        dacc = [None, None]
        nacc = [None, None]
        for c in range(_C):
            ec = jnp.exp2((seg_ref[0, c, r0:r0 + _CHS, :] - m) * scale)
            k = c % 2
            dacc[k] = ec if dacc[k] is None else dacc[k] + ec
            if c > 0:
                term = ec * w_sref[n_idx, c]
                nacc[k] = term if nacc[k] is None else nacc[k] + term
        fch = (nacc[0] + nacc[1]) / (dacc[0] + dacc[1])
        s_buf[pl.ds(rb * _BR + r0, _CHS), :] = fch

    @pl.when(rb == _NRB - 1)
    def _conv_and_reduce():
        macc = None
        z1 = jnp.zeros((1, _W), jnp.float32)
        z2 = jnp.zeros((2, _W), jnp.float32)
        for k in range(_NCC):
            y0 = k * _CHC
            if k == 0:
                r = [jnp.concatenate([z2, s_buf[0:_CHC - 2, :]], axis=0),
                     jnp.concatenate([z1, s_buf[0:_CHC - 1, :]], axis=0)]
                r += [s_buf[dy - 2:dy - 2 + _CHC, :] for dy in range(2, 5)]
                q = jnp.concatenate([z2, s_buf[0:_CHC + 2, :]], axis=0)
            else:
                r = [s_buf[y0 + dy - 2:y0 + dy - 2 + _CHC, :]
                     for dy in range(5)]
                q = s_buf[y0 - 2:y0 + _CHC + 2, :]
            fm = r[2]
            s3 = (r[1] + fm) + r[3]
            s5 = (s3 + r[0]) + r[4]
            u = s3 + fm
            a = 2.0 * s5 + u
            b = s5 + u
            cvv = _EPS * s5 - _EPS * fm
            gx = (_shift_cols(a, -2) + _shift_cols(b, -1) + cvv
                  - _shift_cols(b, 1) - _shift_cols(a, 2))

            t = [_shift_cols(q, d) for d in (-2, -1, 1, 2)]
            t3 = (t[1] + q) + t[2]
            t5 = (t3 + t[0]) + t[3]
            ut = t3 + q
            h0 = 2.0 * t5 + ut
            h1 = t5 + ut
            gy = (h0[0:_CHC, :] - h0[4:_CHC + 4, :]
                  + h1[1:_CHC + 1, :] - h1[3:_CHC + 3, :]
                  + _EPS * t5[2:_CHC + 2, :])

            mag = jnp.sqrt(gx * gx + gy * gy + 1e-08)
            macc = mag if macc is None else macc + mag
        part = jnp.sum(macc)

        @pl.when(n_idx == 0)
        def _init():
            out_ref[:, :] = part[None, None]

        @pl.when(n_idx > 0)
        def _acc():
            out_ref[:, :] += part[None, None]


def kernel(seg_map, label_with_bg):
    n, c, h, w = seg_map.shape
    wz = label_with_bg.at[:, 0].set(0.0)

    grid_spec = pltpu.PrefetchScalarGridSpec(
        num_scalar_prefetch=1,
        grid=(n, _NRB),
        in_specs=[
            pl.BlockSpec((1, c, _BR, w), lambda i, j, w_sref: (i, 0, j, 0)),
        ],
        out_specs=pl.BlockSpec((1, 1), lambda i, j, w_sref: (0, 0)),
        scratch_shapes=[pltpu.VMEM((_PH, _W), jnp.float32)],
    )

    out = pl.pallas_call(
        _body,
        grid_spec=grid_spec,
        out_shape=jax.ShapeDtypeStruct((1, 1), jnp.float32),
    )(wz, seg_map)
    return out[0, 0] / jnp.float32(n * h * w)


# speedup vs baseline: 1.0225x; 1.0225x over previous
import numpy as np

import jax
import jax.numpy as jnp
from jax.experimental import pallas as pl
from jax.experimental.pallas import tpu as pltpu

_BETA = 1000.0
_LOG2E = 1.4426950408889634
_N, _C, _H, _W = 4, 21, 512, 512
_BR = 256
_NRB = _H // _BR
_PH = 520

_CHS = 16
_NCS = _BR // _CHS
_CHC = 8
_NCC = _H // _CHC

_EPS = 1e-06


def _shift_cols(v, d):
    if d == 0:
        return v
    rows = v.shape[0]
    z = jnp.zeros((rows, abs(d)), jnp.float32)
    if d > 0:
        return jnp.concatenate([v[:, d:], z], axis=1)
    return jnp.concatenate([z, v[:, :d]], axis=1)


def _body(w_sref, seg_ref, out_ref, s_buf):
    n_idx = pl.program_id(0)
    rb = pl.program_id(1)
    scale = jnp.float32(_BETA * _LOG2E)

    @pl.when(rb == 0)
    def _zero_halo():
        s_buf[0:2, :] = jnp.zeros((2, _W), jnp.float32)
        s_buf[_H + 2:_PH, :] = jnp.zeros((_PH - _H - 2, _W), jnp.float32)

    for ch in range(_NCS):
        r0 = ch * _CHS
        mx = [seg_ref[0, c, r0:r0 + _CHS, :] for c in range(4)]
        for c in range(4, _C):
            mx[c % 4] = jnp.maximum(mx[c % 4], seg_ref[0, c, r0:r0 + _CHS, :])
        m = jnp.maximum(jnp.maximum(mx[0], mx[1]), jnp.maximum(mx[2], mx[3]))
        dacc = [None, None]
        nacc = [None, None]
        for c in range(_C):
            ec = jnp.exp2((seg_ref[0, c, r0:r0 + _CHS, :] - m) * scale)
            k = c % 2
            dacc[k] = ec if dacc[k] is None else dacc[k] + ec
            if c > 0:
                term = ec * w_sref[n_idx, c]
                nacc[k] = term if nacc[k] is None else nacc[k] + term
        fch = (nacc[0] + nacc[1]) / (dacc[0] + dacc[1])
        for i in range(_NRB):
            @pl.when(rb == i)
            def _store(i=i, r0=r0, fch=fch):
                s_buf[2 + i * _BR + r0:2 + i * _BR + r0 + _CHS, :] = fch

    @pl.when(rb == _NRB - 1)
    def _conv_and_reduce():
        macc = None
        for k in range(_NCC):
            y0 = k * _CHC
            r = [s_buf[y0 + dy:y0 + dy + _CHC, :] for dy in range(5)]
            q = s_buf[y0:y0 + _CHC + 4, :]
            fm = r[2]
            s3 = (r[1] + fm) + r[3]
            s5 = (s3 + r[0]) + r[4]
            u = s3 + fm
            a = 2.0 * s5 + u
            b = s5 + u
            cvv = _EPS * s5 - _EPS * fm
            gx = (_shift_cols(a, -2) + _shift_cols(b, -1) + cvv
                  - _shift_cols(b, 1) - _shift_cols(a, 2))

            t = [_shift_cols(q, d) for d in (-2, -1, 1, 2)]
            t3 = (t[1] + q) + t[2]
            t5 = (t3 + t[0]) + t[3]
            ut = t3 + q
            h0 = 2.0 * t5 + ut
            h1 = t5 + ut
            gy = (h0[0:_CHC, :] - h0[4:_CHC + 4, :]
                  + h1[1:_CHC + 1, :] - h1[3:_CHC + 3, :]
                  + _EPS * t5[2:_CHC + 2, :])

            mag = jnp.sqrt(gx * gx + gy * gy + 1e-08)
            macc = mag if macc is None else macc + mag
        part = jnp.sum(macc)

        @pl.when(n_idx == 0)
        def _init():
            out_ref[:, :] = part[None, None]

        @pl.when(n_idx > 0)
        def _acc():
            out_ref[:, :] += part[None, None]


def kernel(seg_map, label_with_bg):
    n, c, h, w = seg_map.shape
    wz = label_with_bg.at[:, 0].set(0.0)

    grid_spec = pltpu.PrefetchScalarGridSpec(
        num_scalar_prefetch=1,
        grid=(n, _NRB),
        in_specs=[
            pl.BlockSpec((1, c, _BR, w), lambda i, j, w_sref: (i, 0, j, 0)),
        ],
        out_specs=pl.BlockSpec((1, 1), lambda i, j, w_sref: (0, 0)),
        scratch_shapes=[pltpu.VMEM((_PH, _W), jnp.float32)],
    )

    out = pl.pallas_call(
        _body,
        grid_spec=grid_spec,
        out_shape=jax.ShapeDtypeStruct((1, 1), jnp.float32),
    )(wz, seg_map)
    return out[0, 0] / jnp.float32(n * h * w)
